# baseline (speedup 1.0000x reference)
import functools

import jax
import jax.numpy as jnp
from jax import lax
from jax.experimental import pallas as pl
from jax.experimental.pallas import tpu as pltpu
from jax.experimental.pallas import tpu_sc as plsc


def _make_gather(n, v, d, nc, ns):
    nw = nc * ns
    per_w = n // nw
    ch = 64
    n_ch = per_w // ch
    nbuf = 10
    npair = nbuf // 5
    assert n_ch % nbuf == 0

    mesh = plsc.VectorSubcoreMesh(core_axis_name="c", subcore_axis_name="s")

    @functools.partial(
        pl.kernel,
        out_type=jax.ShapeDtypeStruct((n, d), jnp.float32),
        mesh=mesh,
        scratch_types=(
            [pltpu.VMEM((n_ch, ch), jnp.int32),
             pltpu.VMEM((npair, 5 * ch, d), jnp.float32)]
            + [pltpu.SemaphoreType.DMA] * (nbuf + npair)
        ),
    )
    def k(ids_hbm, table_hbm, out_hbm, idx_v, bufs, *sems):
        gsem = sems[:nbuf]
        ssem = sems[nbuf:]
        wid = lax.axis_index("s") * nc + lax.axis_index("c")
        base = wid * per_w
        pltpu.sync_copy(ids_hbm.at[wid], idx_v)

        def gather(p, h, j):
            pltpu.async_copy(table_hbm.at[idx_v.at[j]],
                             bufs.at[p, pl.ds(h * ch, ch)], gsem[5 * p + h])

        def gwait(p, h, j):
            pltpu.make_async_copy(table_hbm.at[idx_v.at[j]],
                                  bufs.at[p, pl.ds(h * ch, ch)],
                                  gsem[5 * p + h]).wait()

        def store(p, j):
            pltpu.async_copy(bufs.at[p], out_hbm.at[pl.ds(base + j * ch, 5 * ch)],
                             ssem[p])

        def swait(p, j):
            pltpu.make_async_copy(bufs.at[p],
                                  out_hbm.at[pl.ds(base + j * ch, 5 * ch)],
                                  ssem[p]).wait()

        for p in range(npair):
            for h in range(5):
                gather(p, h, 5 * p + h)

        def body(g, carry):
            c0 = g * nbuf
            for p in range(npair):
                for h in range(5):
                    gwait(p, h, c0 + 5 * p + h)
                store(p, c0 + 5 * p)
            for p in range(npair):
                swait(p, c0 + 5 * p)
                for h in range(5):
                    @pl.when(c0 + 5 * p + h + nbuf < n_ch)
                    def _():
                        gather(p, h, c0 + 5 * p + h + nbuf)

            return carry

        lax.fori_loop(0, n_ch // nbuf, body, 0)

    return k


def kernel(tok_ids, table):
    b, l = tok_ids.shape
    v, d = table.shape
    n = b * l
    nc, ns = 2, 16
    ids = tok_ids.reshape(nc * ns, n // (nc * ns) // 64, 64)
    out = _make_gather(n, v, d, nc, ns)(ids, table)
    return out.reshape(b, l, d)
